# Initial kernel scaffold; baseline (speedup 1.0000x reference)
import functools

import jax
import jax.numpy as jnp
from jax import lax
from jax.experimental import pallas as pl
from jax.experimental.pallas import tpu as pltpu
from jax.experimental.pallas import tpu_sc as plsc

N = 10000
E = 320000
D = 128

NC = 2
NS = 16
NW = NC * NS
C = 80
EPT = E // NW
NCHUNK = EPT // C
ROWS_PT = N // NS
ZROWS = 125


def _mm_body(x_ref, w_ref, o_ref):
    o_ref[...] = jnp.dot(x_ref[...], w_ref[...],
                         preferred_element_type=jnp.float32)


def _matmul(x, w):
    MB = 1000
    return pl.pallas_call(
        _mm_body,
        grid=(N // MB,),
        in_specs=[pl.BlockSpec((MB, D), lambda i: (i, 0)),
                  pl.BlockSpec((D, D), lambda i: (0, 0))],
        out_specs=pl.BlockSpec((MB, D), lambda i: (i, 0)),
        out_shape=jax.ShapeDtypeStruct((N, D), jnp.float32),
    )(x, w)


def _comb_body(p_ref, b_ref, o_ref):
    o_ref[...] = p_ref[0] + p_ref[1] + b_ref[0:1]


def _combine(partials, bias):
    MB = 1000
    bias8 = jnp.broadcast_to(bias.reshape(1, D), (8, D))
    return pl.pallas_call(
        _comb_body,
        grid=(N // MB,),
        in_specs=[pl.BlockSpec((2, MB, D), lambda i: (0, i, 0)),
                  pl.BlockSpec((8, D), lambda i: (0, 0))],
        out_specs=pl.BlockSpec((MB, D), lambda i: (i, 0)),
        out_shape=jax.ShapeDtypeStruct((N, D), jnp.float32),
    )(partials, bias8)


_mesh = plsc.VectorSubcoreMesh(core_axis_name="c", subcore_axis_name="s")


@functools.partial(
    pl.kernel,
    out_type=jax.ShapeDtypeStruct((NC, N, D), jnp.float32),
    mesh=_mesh,
    scratch_types=[
        pltpu.VMEM_SHARED((N, D), jnp.float32),
        pltpu.VMEM((NCHUNK, C), jnp.int32),
        pltpu.VMEM((NCHUNK, C), jnp.int32),
        pltpu.VMEM((NCHUNK, C), jnp.float32),
        pltpu.VMEM((C, D), jnp.float32),
        pltpu.VMEM((ZROWS, D), jnp.float32),
        pltpu.SemaphoreType.DMA,
    ],
)
def _sc_edges(sup, row2, col2, ew2, out,
              acc, row_v, col_v, w_v, rows_v, zbuf, sem):
    c = lax.axis_index("c")
    s = lax.axis_index("s")
    wid = s * NC + c

    zero16 = jnp.zeros((16,), jnp.float32)

    def _zrow(r, _):
        for j in range(D // 16):
            zbuf[r, pl.ds(j * 16, 16)] = zero16
        return 0

    lax.fori_loop(0, ZROWS, _zrow, 0)
    for k in range(ROWS_PT // ZROWS):
        pltpu.sync_copy(zbuf, acc.at[pl.ds(s * ROWS_PT + k * ZROWS, ZROWS)])
    plsc.subcore_barrier()

    pltpu.sync_copy(row2.at[pl.ds(wid * NCHUNK, NCHUNK)], row_v)
    pltpu.sync_copy(col2.at[pl.ds(wid * NCHUNK, NCHUNK)], col_v)
    pltpu.sync_copy(ew2.at[pl.ds(wid * NCHUNK, NCHUNK)], w_v)

    def _chunk(k, _):
        pltpu.async_copy(sup.at[row_v.at[k]], rows_v, sem).wait()
        for e in range(C):
            w_e = w_v[k, e]
            for j in range(D // 16):
                sl = pl.ds(j * 16, 16)
                rows_v[e, sl] = rows_v[e, sl] * w_e
        pltpu.sync_copy(rows_v, acc.at[col_v.at[k]], add=True)
        return 0

    lax.fori_loop(0, NCHUNK, _chunk, 0)
    plsc.subcore_barrier()

    pltpu.sync_copy(acc.at[pl.ds(s * ROWS_PT, ROWS_PT)],
                    out.at[c, pl.ds(s * ROWS_PT, ROWS_PT)])


def kernel(x, edge_index, edge_weight, weight, bias):
    row = edge_index[0].astype(jnp.int32).reshape(E // C, C)
    col = edge_index[1].astype(jnp.int32).reshape(E // C, C)
    ew = edge_weight.reshape(E // C, C)
    support = _matmul(x, weight)
    partials = _sc_edges(support, row, col, ew)
    return _combine(partials, bias)


# speedup vs baseline: 5.6540x; 5.6540x over previous
import functools

import jax
import jax.numpy as jnp
from jax import lax
from jax.experimental import pallas as pl
from jax.experimental.pallas import tpu as pltpu
from jax.experimental.pallas import tpu_sc as plsc

N = 10000
E = 320000
D = 128

NC = 2
NS = 16
NW = NC * NS
C = 80
EPT = E // NW
NPASS = 5
PCHUNK = EPT // C // NPASS
N_PAD = 10240
ROWS_PT = N_PAD // NS


def _mm_body(x_ref, w_ref, o_ref):
    o_ref[...] = jnp.dot(x_ref[...], w_ref[...],
                         preferred_element_type=jnp.float32)


def _matmul(x, w):
    MB = 1000
    return pl.pallas_call(
        _mm_body,
        grid=(N // MB,),
        in_specs=[pl.BlockSpec((MB, D), lambda i: (i, 0)),
                  pl.BlockSpec((D, D), lambda i: (0, 0))],
        out_specs=pl.BlockSpec((MB, D), lambda i: (i, 0)),
        out_shape=jax.ShapeDtypeStruct((N, D), jnp.float32),
    )(x, w)


def _comb_body(p_ref, b_ref, o_ref):
    o_ref[...] = p_ref[0] + p_ref[1] + b_ref[0:1]


def _combine(partials, bias):
    MB = 1000
    bias8 = jnp.broadcast_to(bias.reshape(1, D), (8, D))
    return pl.pallas_call(
        _comb_body,
        grid=(N // MB,),
        in_specs=[pl.BlockSpec((2, MB, D), lambda i: (0, i, 0)),
                  pl.BlockSpec((8, D), lambda i: (0, 0))],
        out_specs=pl.BlockSpec((MB, D), lambda i: (i, 0)),
        out_shape=jax.ShapeDtypeStruct((N, D), jnp.float32),
    )(partials, bias8)


_mesh = plsc.VectorSubcoreMesh(core_axis_name="c", subcore_axis_name="s")


@functools.partial(
    pl.kernel,
    out_type=jax.ShapeDtypeStruct((NC, N_PAD, D), jnp.float32),
    mesh=_mesh,
    scratch_types=[
        pltpu.VMEM_SHARED((N_PAD, D), jnp.float32),
        pltpu.VMEM((PCHUNK, 2, C), jnp.int32),
        pltpu.VMEM((PCHUNK, C), jnp.float32),
        pltpu.VMEM((C, D), jnp.float32),
        pltpu.SemaphoreType.DMA,
    ],
)
def _sc_edges(sup, rc, ew, out, acc, rc_v, w_v, rows_v, sem):
    c = lax.axis_index("c")
    s = lax.axis_index("s")
    wid = s * NC + c

    zero16 = jnp.zeros((16,), jnp.float32)

    def _zrow(r, _):
        for j in range(D // 16):
            rows_v[r, pl.ds(j * 16, 16)] = zero16
        return 0

    lax.fori_loop(0, C, _zrow, 0)
    for k in range(ROWS_PT // C):
        pltpu.sync_copy(rows_v, acc.at[pl.ds(s * ROWS_PT + k * C, C)])
    plsc.subcore_barrier()

    def _chunk(k, _):
        pltpu.async_copy(sup.at[rc_v.at[k, 0]], rows_v, sem).wait()
        for g in range(C // 16):
            wvec = w_v[k, pl.ds(g * 16, 16)]
            for t in range(16):
                e = g * 16 + t
                w_e = wvec[t]
                for j in range(D // 16):
                    sl = pl.ds(j * 16, 16)
                    rows_v[e, sl] = rows_v[e, sl] * w_e
        pltpu.sync_copy(rows_v, acc.at[rc_v.at[k, 1]], add=True)
        return 0

    for p in range(NPASS):
        pltpu.sync_copy(rc.at[wid, p], rc_v)
        pltpu.sync_copy(ew.at[wid, p], w_v)
        lax.fori_loop(0, PCHUNK, _chunk, 0)
    plsc.subcore_barrier()

    pltpu.sync_copy(acc.at[pl.ds(s * ROWS_PT, ROWS_PT)],
                    out.at[c, pl.ds(s * ROWS_PT, ROWS_PT)])


def kernel(x, edge_index, edge_weight, weight, bias):
    row = edge_index[0].astype(jnp.int32).reshape(NW, NPASS, PCHUNK, C)
    col = edge_index[1].astype(jnp.int32).reshape(NW, NPASS, PCHUNK, C)
    rc = jnp.stack([row, col], axis=3)
    ew = edge_weight.reshape(NW, NPASS, PCHUNK, C)
    support = _matmul(x, weight)
    partials = _sc_edges(support, rc, ew)
    return _combine(partials, bias)
